# Initial kernel scaffold; baseline (speedup 1.0000x reference)
import jax
import jax.numpy as jnp
from jax.experimental import pallas as pl


def kernel(input_ids, attention_mask, token_emb, pos_emb, W_expert, W_memory, W_budget, mem_keys, mem_values, Wq, Wo_mem, W1, b1, W2, b2, W_out, b_out, ln_g, ln_b, W_lm, b_lm):
    raise NotImplementedError("write your pallas kernel here")


# speedup vs baseline: 1.0731x; 1.0731x over previous
import functools
import math

import jax
import jax.numpy as jnp
from jax import lax
from jax.experimental import pallas as pl
from jax.experimental.pallas import tpu as pltpu
from jax.experimental.pallas import tpu_sc as plsc

B, S = 4, 512
N = B * S
V = 32000
D = 1024
E = 8
M = 64
H_MEM = 8
DH = D // H_MEM
H_FF = D // 2
TEMP = 1.0
TARGET_BUDGET = 0.3
BUDGET_W = 0.01
EPS = 1e-5

T = 256
N_TILES = N // T
VT = 1280
V_TILES = V // VT

_f32 = jnp.float32
_bf16 = jnp.bfloat16


def _emb_gather(table, ids):
    info = plsc.get_sparse_core_info()
    nc, ns = info.num_cores, info.num_subcores
    nw = nc * ns
    b_per_w = N // nw
    mesh = plsc.VectorSubcoreMesh(core_axis_name="c", subcore_axis_name="s")

    @functools.partial(
        pl.kernel,
        mesh=mesh,
        out_type=jax.ShapeDtypeStruct((N, D), _f32),
        scratch_types=[
            pltpu.VMEM((b_per_w,), jnp.int32),
            pltpu.VMEM((b_per_w, D), _f32),
            pltpu.SemaphoreType.DMA,
        ],
    )
    def gather_kernel(table_hbm, idx_hbm, out_hbm, idx_v, rows_v, sem):
        wid = lax.axis_index("s") * nc + lax.axis_index("c")
        base = wid * b_per_w
        pltpu.sync_copy(idx_hbm.at[pl.ds(base, b_per_w)], idx_v)
        pltpu.async_copy(table_hbm.at[idx_v], rows_v, sem).wait()
        pltpu.sync_copy(rows_v, out_hbm.at[pl.ds(base, b_per_w)])

    return gather_kernel(table, ids)


def _softmax(z):
    zm = z - jnp.max(z, axis=-1, keepdims=True)
    e = jnp.exp(zm)
    return e / jnp.sum(e, axis=-1, keepdims=True)


def _trunk_body(x_ref, pos_ref, mask_ref, we_ref, wm_ref, wb_ref, wq_ref,
                kT_ref, v_ref, wo_ref, w1_ref, b1_ref, w2_ref, b2_ref,
                wout_ref, bout_ref, lng_ref, lnb_ref, xo_ref, stats_ref):
    i = pl.program_id(0)
    x = (x_ref[...] + pos_ref[...]) * mask_ref[...]

    el = jnp.dot(x, we_ref[...], preferred_element_type=_f32)
    p = _softmax(el / TEMP)
    m1 = jnp.max(p, axis=-1, keepdims=True)
    p_wo = jnp.where(p == m1, -1.0, p)
    m2 = jnp.max(p_wo, axis=-1, keepdims=True)
    emask = (p >= m2).astype(_f32)
    ew = p * emask
    ew = ew / (jnp.sum(ew, axis=-1, keepdims=True) + 1e-9)

    ml = jnp.dot(x, wm_ref[...], preferred_element_type=_f32)
    mp = _softmax(ml)
    mmask = (mp > (1.0 / M)).astype(_f32)
    mw = mp * mmask
    mw = mw / (jnp.sum(mw, axis=-1, keepdims=True) + 1e-9)

    bl = jnp.dot(x, wb_ref[...], preferred_element_type=_f32)
    pred = 1.0 / (1.0 + jnp.exp(-bl))

    xb = x.astype(_bf16)
    q = jnp.dot(xb, wq_ref[...], preferred_element_type=_f32)
    qb = q.astype(_bf16)
    scale = 1.0 / math.sqrt(DH)
    ctx_heads = []
    for h in range(H_MEM):
        sl = slice(h * DH, (h + 1) * DH)
        sc = jnp.dot(qb[:, sl], kT_ref[sl, :],
                     preferred_element_type=_f32) * scale
        a = _softmax(sc)
        a = a * mw
        a = a / (jnp.sum(a, axis=-1, keepdims=True) + 1e-9)
        ctx_heads.append(jnp.dot(a.astype(_bf16), v_ref[:, sl],
                                 preferred_element_type=_f32))
    ctx = jnp.concatenate(ctx_heads, axis=1)
    combined = jnp.dot(ctx.astype(_bf16), wo_ref[...],
                       preferred_element_type=_f32)

    for e in range(E):
        he = jnp.dot(xb, w1_ref[e], preferred_element_type=_f32)
        he = jax.nn.gelu(he + b1_ref[e:e + 1, :])
        ye = jnp.dot(he.astype(_bf16), w2_ref[e],
                     preferred_element_type=_f32) + b2_ref[e:e + 1, :]
        combined = combined + ew[:, e:e + 1] * ye

    out = jnp.dot(combined.astype(_bf16), wout_ref[...],
                  preferred_element_type=_f32) + bout_ref[...]
    mu = jnp.mean(out, axis=-1, keepdims=True)
    var = jnp.mean((out - mu) ** 2, axis=-1, keepdims=True)
    o = (out - mu) * lax.rsqrt(var + EPS) * lng_ref[...] + lnb_ref[...]
    xo_ref[...] = o.astype(_bf16)

    eu = jnp.sum(emask, axis=-1) / E
    mu_use = jnp.sum(mmask, axis=-1) / M
    actual = 0.5 * (eu + mu_use)
    s1 = jnp.sum((actual - TARGET_BUDGET) ** 2)
    s2 = jnp.sum((pred[:, 0] - actual) ** 2)
    s3 = jnp.sum(mu_use)
    s4 = jnp.sum(eu)
    lane = lax.broadcasted_iota(jnp.int32, (1, 128), 1)
    row = (jnp.where(lane == 0, s1, 0.0) + jnp.where(lane == 1, s2, 0.0)
           + jnp.where(lane == 2, s3, 0.0) + jnp.where(lane == 3, s4, 0.0))

    @pl.when(i == 0)
    def _():
        stats_ref[...] = jnp.zeros_like(stats_ref)

    stats_ref[...] += row


def _trunk(x_raw, pos, maskc, W_expert, W_memory, W_budget, Wq_bf, kT_bf,
           v_bf, Wo_bf, W1_bf, b1, W2_bf, b2, Wout_bf, bout, lng, lnb):
    full = lambda a: pl.BlockSpec(a.shape, lambda i: (0,) * a.ndim)
    return pl.pallas_call(
        _trunk_body,
        grid=(N_TILES,),
        in_specs=[
            pl.BlockSpec((T, D), lambda i: (i, 0)),
            pl.BlockSpec((T, D), lambda i: (i % (S // T), 0)),
            pl.BlockSpec((T, 1), lambda i: (i, 0)),
            full(W_expert), full(W_memory), full(W_budget), full(Wq_bf),
            full(kT_bf), full(v_bf), full(Wo_bf), full(W1_bf), full(b1),
            full(W2_bf), full(b2), full(Wout_bf), full(bout), full(lng),
            full(lnb),
        ],
        out_specs=[
            pl.BlockSpec((T, D), lambda i: (i, 0)),
            pl.BlockSpec((1, 128), lambda i: (0, 0)),
        ],
        out_shape=[
            jax.ShapeDtypeStruct((N, D), _bf16),
            jax.ShapeDtypeStruct((1, 128), _f32),
        ],
    )(x_raw, pos, maskc, W_expert, W_memory, W_budget, Wq_bf, kT_bf, v_bf,
      Wo_bf, W1_bf, b1, W2_bf, b2, Wout_bf, bout, lng, lnb)


def _head_body(x_ref, w_ref, b_ref, o_ref):
    o_ref[...] = jnp.dot(x_ref[...], w_ref[...],
                         preferred_element_type=_f32) + b_ref[...]


def _head(xo, Wlm_bf, blm):
    return pl.pallas_call(
        _head_body,
        grid=(V_TILES,),
        in_specs=[
            pl.BlockSpec((N, D), lambda j: (0, 0)),
            pl.BlockSpec((D, VT), lambda j: (0, j)),
            pl.BlockSpec((1, VT), lambda j: (0, j)),
        ],
        out_specs=pl.BlockSpec((N, VT), lambda j: (0, j)),
        out_shape=jax.ShapeDtypeStruct((N, V), _f32),
    )(xo, Wlm_bf, blm)


def kernel(input_ids, attention_mask, token_emb, pos_emb, W_expert, W_memory,
           W_budget, mem_keys, mem_values, Wq, Wo_mem, W1, b1, W2, b2, W_out,
           b_out, ln_g, ln_b, W_lm, b_lm):
    ids = input_ids.reshape(N).astype(jnp.int32)
    x_raw = _emb_gather(token_emb, ids)

    pos = pos_emb[:S]
    maskc = attention_mask.reshape(N, 1)
    xo, stats = _trunk(
        x_raw, pos, maskc, W_expert, W_memory, W_budget,
        Wq.astype(_bf16), mem_keys.T.astype(_bf16), mem_values.astype(_bf16),
        Wo_mem.astype(_bf16), W1.astype(_bf16), b1, W2.astype(_bf16), b2,
        W_out.astype(_bf16), b_out.reshape(1, D), ln_g.reshape(1, D),
        ln_b.reshape(1, D))

    logits = _head(xo, W_lm.astype(_bf16), b_lm.reshape(1, V)).reshape(B, S, V)

    s1, s2, s3, s4 = stats[0, 0], stats[0, 1], stats[0, 2], stats[0, 3]
    budget_loss = BUDGET_W * (s1 / N + s2 / N)
    return logits, budget_loss, s4 / N, s3 / N


# speedup vs baseline: 1.3447x; 1.2531x over previous
import functools
import math

import jax
import jax.numpy as jnp
from jax import lax
from jax.experimental import pallas as pl
from jax.experimental.pallas import tpu as pltpu
from jax.experimental.pallas import tpu_sc as plsc

B, S = 4, 512
N = B * S
V = 32000
D = 1024
E = 8
M = 64
H_MEM = 8
DH = D // H_MEM
H_FF = D // 2
TEMP = 1.0
TARGET_BUDGET = 0.3
BUDGET_W = 0.01
EPS = 1e-5

T = 512
N_TILES = N // T
VT = 1280
V_TILES = V // VT

_f32 = jnp.float32
_bf16 = jnp.bfloat16


def _emb_gather(table, ids):
    info = plsc.get_sparse_core_info()
    nc, ns = info.num_cores, info.num_subcores
    nw = nc * ns
    b_per_w = N // nw
    mesh = plsc.VectorSubcoreMesh(core_axis_name="c", subcore_axis_name="s")

    @functools.partial(
        pl.kernel,
        mesh=mesh,
        out_type=jax.ShapeDtypeStruct((N, D), _f32),
        scratch_types=[
            pltpu.VMEM((b_per_w,), jnp.int32),
            pltpu.VMEM((b_per_w, D), _f32),
            pltpu.SemaphoreType.DMA,
        ],
    )
    def gather_kernel(table_hbm, idx_hbm, out_hbm, idx_v, rows_v, sem):
        wid = lax.axis_index("s") * nc + lax.axis_index("c")
        base = wid * b_per_w
        pltpu.sync_copy(idx_hbm.at[pl.ds(base, b_per_w)], idx_v)
        pltpu.async_copy(table_hbm.at[idx_v], rows_v, sem).wait()
        pltpu.sync_copy(rows_v, out_hbm.at[pl.ds(base, b_per_w)])

    return gather_kernel(table, ids)


def _softmax(z):
    zm = z - jnp.max(z, axis=-1, keepdims=True)
    e = jnp.exp(zm)
    return e / jnp.sum(e, axis=-1, keepdims=True)


def _trunk_body(x_ref, pos_ref, mask_ref, we_ref, wm_ref, wb_ref, wq_ref,
                kT_ref, v_ref, wo_ref, w1_ref, b1_ref, w2_ref, b2_ref,
                wout_ref, bout_ref, lng_ref, lnb_ref, xo_ref, stats_ref):
    i = pl.program_id(0)
    x = (x_ref[...] + pos_ref[...]) * mask_ref[...]

    el = jnp.dot(x, we_ref[...], preferred_element_type=_f32)
    p = _softmax(el / TEMP)
    m1 = jnp.max(p, axis=-1, keepdims=True)
    p_wo = jnp.where(p == m1, -1.0, p)
    m2 = jnp.max(p_wo, axis=-1, keepdims=True)
    emask = (p >= m2).astype(_f32)
    ew = p * emask
    ew = ew / (jnp.sum(ew, axis=-1, keepdims=True) + 1e-9)

    ml = jnp.dot(x, wm_ref[...], preferred_element_type=_f32)
    mp = _softmax(ml)
    mmask = (mp > (1.0 / M)).astype(_f32)
    mw = mp * mmask
    mw = mw / (jnp.sum(mw, axis=-1, keepdims=True) + 1e-9)

    bl = jnp.dot(x, wb_ref[...], preferred_element_type=_f32)
    pred = 1.0 / (1.0 + jnp.exp(-bl))

    xb = x.astype(_bf16)
    q = jnp.dot(xb, wq_ref[...], preferred_element_type=_f32)
    qb = q.astype(_bf16)
    scale = 1.0 / math.sqrt(DH)
    ctx_heads = []
    for h in range(H_MEM):
        sl = slice(h * DH, (h + 1) * DH)
        sc = jnp.dot(qb[:, sl], kT_ref[sl, :],
                     preferred_element_type=_f32) * scale
        a = _softmax(sc)
        a = a * mw
        a = a / (jnp.sum(a, axis=-1, keepdims=True) + 1e-9)
        ctx_heads.append(jnp.dot(a.astype(_bf16), v_ref[:, sl],
                                 preferred_element_type=_f32))
    ctx = jnp.concatenate(ctx_heads, axis=1)
    combined = jnp.dot(ctx.astype(_bf16), wo_ref[...],
                       preferred_element_type=_f32)

    for e in range(E):
        he = jnp.dot(xb, w1_ref[e], preferred_element_type=_f32)
        he = jax.nn.gelu(he + b1_ref[e:e + 1, :])
        ye = jnp.dot(he.astype(_bf16), w2_ref[e],
                     preferred_element_type=_f32) + b2_ref[e:e + 1, :]
        combined = combined + ew[:, e:e + 1] * ye

    out = jnp.dot(combined.astype(_bf16), wout_ref[...],
                  preferred_element_type=_f32) + bout_ref[...]
    mu = jnp.mean(out, axis=-1, keepdims=True)
    var = jnp.mean((out - mu) ** 2, axis=-1, keepdims=True)
    o = (out - mu) * lax.rsqrt(var + EPS) * lng_ref[...] + lnb_ref[...]
    xo_ref[...] = o.astype(_bf16)

    eu = jnp.sum(emask, axis=-1) / E
    mu_use = jnp.sum(mmask, axis=-1) / M
    actual = 0.5 * (eu + mu_use)
    s1 = jnp.sum((actual - TARGET_BUDGET) ** 2)
    s2 = jnp.sum((pred[:, 0] - actual) ** 2)
    s3 = jnp.sum(mu_use)
    s4 = jnp.sum(eu)
    lane = lax.broadcasted_iota(jnp.int32, (1, 128), 1)
    row = (jnp.where(lane == 0, s1, 0.0) + jnp.where(lane == 1, s2, 0.0)
           + jnp.where(lane == 2, s3, 0.0) + jnp.where(lane == 3, s4, 0.0))

    @pl.when(i == 0)
    def _():
        stats_ref[...] = jnp.zeros_like(stats_ref)

    stats_ref[...] += row


def _trunk(x_raw, pos, maskc, W_expert, W_memory, W_budget, Wq_bf, kT_bf,
           v_bf, Wo_bf, W1_bf, b1, W2_bf, b2, Wout_bf, bout, lng, lnb):
    full = lambda a: pl.BlockSpec(a.shape, lambda i: (0,) * a.ndim)
    return pl.pallas_call(
        _trunk_body,
        grid=(N_TILES,),
        in_specs=[
            pl.BlockSpec((T, D), lambda i: (i, 0)),
            pl.BlockSpec((T, D), lambda i: (i % (S // T), 0)),
            pl.BlockSpec((T, 1), lambda i: (i, 0)),
            full(W_expert), full(W_memory), full(W_budget), full(Wq_bf),
            full(kT_bf), full(v_bf), full(Wo_bf), full(W1_bf), full(b1),
            full(W2_bf), full(b2), full(Wout_bf), full(bout), full(lng),
            full(lnb),
        ],
        out_specs=[
            pl.BlockSpec((T, D), lambda i: (i, 0)),
            pl.BlockSpec((1, 128), lambda i: (0, 0)),
        ],
        out_shape=[
            jax.ShapeDtypeStruct((N, D), _bf16),
            jax.ShapeDtypeStruct((1, 128), _f32),
        ],
    )(x_raw, pos, maskc, W_expert, W_memory, W_budget, Wq_bf, kT_bf, v_bf,
      Wo_bf, W1_bf, b1, W2_bf, b2, Wout_bf, bout, lng, lnb)


def _head_body(x_ref, w_ref, b_ref, o_ref):
    o_ref[...] = jnp.dot(x_ref[...], w_ref[...].astype(_bf16),
                         preferred_element_type=_f32) + b_ref[...]


def _head(xo, Wlm_bf, blm):
    return pl.pallas_call(
        _head_body,
        grid=(V_TILES,),
        in_specs=[
            pl.BlockSpec((N, D), lambda j: (0, 0)),
            pl.BlockSpec((D, VT), lambda j: (0, j)),
            pl.BlockSpec((1, VT), lambda j: (0, j)),
        ],
        out_specs=pl.BlockSpec((N, VT), lambda j: (0, j)),
        out_shape=jax.ShapeDtypeStruct((N, V), _f32),
    )(xo, Wlm_bf, blm)


def kernel(input_ids, attention_mask, token_emb, pos_emb, W_expert, W_memory,
           W_budget, mem_keys, mem_values, Wq, Wo_mem, W1, b1, W2, b2, W_out,
           b_out, ln_g, ln_b, W_lm, b_lm):
    ids = input_ids.reshape(N).astype(jnp.int32)
    x_raw = _emb_gather(token_emb, ids)

    pos = pos_emb[:S]
    maskc = attention_mask.reshape(N, 1)
    xo, stats = _trunk(
        x_raw, pos, maskc, W_expert, W_memory, W_budget,
        Wq.astype(_bf16), mem_keys.T.astype(_bf16), mem_values.astype(_bf16),
        Wo_mem.astype(_bf16), W1.astype(_bf16), b1, W2.astype(_bf16), b2,
        W_out.astype(_bf16), b_out.reshape(1, D), ln_g.reshape(1, D),
        ln_b.reshape(1, D))

    logits = _head(xo, W_lm, b_lm.reshape(1, V)).reshape(B, S, V)

    s1, s2, s3, s4 = stats[0, 0], stats[0, 1], stats[0, 2], stats[0, 3]
    budget_loss = BUDGET_W * (s1 / N + s2 / N)
    return logits, budget_loss, s4 / N, s3 / N
